# baseline (speedup 1.0000x reference)
import functools

import jax
import jax.numpy as jnp
from jax import lax
from jax.experimental import pallas as pl
from jax.experimental.pallas import tpu as pltpu
from jax.experimental.pallas import tpu_sc as plsc

GNN = 128
RNN = 256
HEADS = 4
CHUNK = 2000
LANES = 16
SC_CORES = 2
SC_SUBCORES = 16
SC_WORKERS = SC_CORES * SC_SUBCORES
MROWS = 80
NPAD = MROWS * GNN
CAP_T = 16
CAP_C = CAP_T * SC_SUBCORES
NSP = SC_CORES * CAP_C


def _sc_multiplicity(edge_index, nf):
    E = edge_index.shape[1]
    tiles = E // 128
    tpw = tiles // SC_WORKERS
    extra = tiles - tpw * SC_WORKERS
    epw = tpw * 128
    mesh = plsc.VectorSubcoreMesh(core_axis_name="c", subcore_axis_name="s")

    @functools.partial(
        pl.kernel,
        out_type=(
            jax.ShapeDtypeStruct((SC_CORES, MROWS, GNN), jnp.float32),
            jax.ShapeDtypeStruct((SC_CORES, CAP_C, GNN), jnp.float32),
            jax.ShapeDtypeStruct((SC_CORES, CAP_C), jnp.float32),
            jax.ShapeDtypeStruct((SC_CORES, SC_SUBCORES, LANES), jnp.int32),
        ),
        mesh=mesh,
        scratch_types=[
            pltpu.VMEM((2, epw + 128), jnp.int32),
            pltpu.VMEM((MROWS, GNN), jnp.float32),
            pltpu.VMEM((MROWS,), jnp.int32),
            pltpu.VMEM_SHARED((MROWS, GNN), jnp.float32),
            pltpu.VMEM((2 * LANES,), jnp.int32),
            pltpu.VMEM((2 * LANES,), jnp.float32),
            pltpu.VMEM((LANES,), jnp.int32),
            pltpu.VMEM((CAP_T, GNN), jnp.float32),
            pltpu.VMEM((LANES,), jnp.int32),
            pltpu.SemaphoreType.DMA,
        ],
        compiler_params=pltpu.CompilerParams(needs_layout_passes=False),
    )
    def k(edge_hbm, nf_hbm, out_hbm, rows_hbm, ml_hbm, k_hbm,
          e_v, m_v, idx_v, shared, sl_v, mlv, sl16_v, rows_v, kv, sem):
        cid = lax.axis_index("c")
        sid = lax.axis_index("s")
        wid = sid * SC_CORES + cid
        base = wid * epw
        pltpu.sync_copy(edge_hbm.at[:, pl.ds(base, epw)],
                        e_v.at[:, pl.ds(0, epw)])

        @pl.when(wid < extra)
        def _():
            xoff = (SC_WORKERS * tpw + wid) * 128
            pltpu.sync_copy(edge_hbm.at[:, pl.ds(xoff, 128)],
                            e_v.at[:, pl.ds(epw, 128)])

        zeros16 = jnp.zeros((LANES,), jnp.float32)
        iota16 = lax.broadcasted_iota(jnp.int32, (LANES,), 0)

        def zero_body(r, carry):
            for j in range(8):
                m_v[r, pl.ds(j * LANES, LANES)] = zeros16
            return carry

        lax.fori_loop(0, MROWS, zero_body, 0)

        def idx_body(j, carry):
            idx_v[pl.ds(j * LANES, LANES)] = iota16 + j * LANES
            return carry

        lax.fori_loop(0, MROWS // LANES, idx_body, 0)

        @pl.when(sid == 0)
        def _():
            pltpu.sync_copy(m_v, shared)

        plsc.subcore_barrier()

        ones16 = jnp.ones((LANES,), jnp.float32)

        def edge_one(i):
            s16 = e_v[0, pl.ds(i * LANES, LANES)]
            d16 = e_v[1, pl.ds(i * LANES, LANES)]
            plsc.addupdate_scatter(
                m_v, [s16 >> 7, s16 & 127], ones16, mask=d16 == 0)

        plsc.parallel_loop(0, epw // LANES, 1, unroll=8)(edge_one)

        @pl.when(wid < extra)
        def _():
            def edge_fori(i, carry):
                edge_one(i)
                return carry
            lax.fori_loop(epw // LANES, (epw + 128) // LANES, edge_fori, 0)

        @pl.when((sid == 0) & (cid == 0))
        def _():
            one0 = jnp.where(iota16 == 0, 1.0, 0.0)
            m_v[0, pl.ds(0, LANES)] = m_v[0, pl.ds(0, LANES)] + one0

        pltpu.sync_copy(m_v, shared.at[idx_v], add=True)
        plsc.subcore_barrier()

        @pl.when(sid == 0)
        def _():
            pltpu.sync_copy(shared, out_hbm.at[cid])

        zero_i16 = jnp.zeros((LANES,), jnp.int32)
        sl_v[pl.ds(0, LANES)] = zero_i16
        sl_v[pl.ds(LANES, LANES)] = zero_i16
        mlv[pl.ds(0, LANES)] = zeros16
        mlv[pl.ds(LANES, LANES)] = zeros16

        def comp(t, off):
            r = t // 8
            j = t - r * 8
            val = m_v[r, pl.ds(j * LANES, LANES)]
            mask = val > 0.0
            mi = jnp.where(mask, 1, 0)
            pos = off + lax.cumsum(mi, axis=0) - 1

            @pl.when(off <= CAP_T)
            def _():
                node = iota16 + (r * GNN + j * LANES)
                plsc.store_scatter(sl_v, [pos], node, mask=mask)
                plsc.store_scatter(mlv, [pos], val, mask=mask)

            return off + jnp.sum(mi)

        kf = lax.fori_loop(0, MROWS * 8, comp, jnp.int32(0))
        kv[...] = jnp.broadcast_to(kf, (LANES,))
        pltpu.sync_copy(kv, k_hbm.at[cid, sid])
        pltpu.sync_copy(mlv.at[pl.ds(0, LANES)],
                        ml_hbm.at[cid, pl.ds(sid * CAP_T, CAP_T)])
        sl16_v[...] = sl_v[pl.ds(0, LANES)]
        pltpu.async_copy(nf_hbm.at[sl16_v], rows_v, sem).wait()
        pltpu.sync_copy(rows_v, rows_hbm.at[cid, pl.ds(sid * CAP_T, CAP_T)])

    return k(edge_index, nf)


def _tc_body(nf_ref, m_ref, hid_ref, Wenc_ref, benc_ref, Wl_ref, bl_ref,
             Wr_ref, br_ref, Ad_ref, Wlad_ref, blad_ref, gb_ref, Wih_ref,
             bih_ref, Whh_ref, bhh_ref, out_ref,
             xr0_s, c1_s, max_s, den_s, num_s, mcol_s):
    s = pl.program_id(0)
    nch = pl.num_programs(0) - 1

    @pl.when(s == 0)
    def _init():
        max_s[...] = jnp.full_like(max_s, -1e30)
        den_s[...] = jnp.zeros_like(den_s)
        num_s[...] = jnp.zeros_like(num_s)
        mgrid = m_ref[0] + m_ref[1]
        rep = jnp.broadcast_to(mgrid[:, None, :], (MROWS, GNN, GNN))
        rep = rep.reshape(NPAD, GNN)
        lane = lax.broadcasted_iota(jnp.int32, (NPAD, GNN), 1)
        rowm = lax.broadcasted_iota(jnp.int32, (NPAD, GNN), 0) & (GNN - 1)
        mcol = jnp.sum(jnp.where(lane == rowm, rep, 0.0), axis=1, keepdims=True)
        mcol_s[...] = mcol

    @pl.when(s < nch)
    def _scan():
        x = jnp.maximum(
            jnp.dot(nf_ref[...], Wenc_ref[...],
                    preferred_element_type=jnp.float32) + benc_ref[...], 0.0)
        xl = jnp.dot(x, Wl_ref[...],
                     preferred_element_type=jnp.float32) + bl_ref[...]

        @pl.when(s == 0)
        def _():
            xr0 = jnp.dot(x[0:1, :], Wr_ref[...],
                          preferred_element_type=jnp.float32) + br_ref[...]
            xr0_s[...] = xr0
            c1_s[...] = jnp.dot(xr0, Ad_ref[...],
                                preferred_element_type=jnp.float32)

        mc = mcol_s[pl.ds(s * CHUNK, CHUNK), :]

        t1 = jnp.dot(x, Wlad_ref[...],
                     preferred_element_type=jnp.float32) + blad_ref[...]
        v = xl + xr0_s[...]
        t2 = jnp.dot(jnp.abs(v), Ad_ref[...],
                     preferred_element_type=jnp.float32)
        alpha = 0.6 * (t1 + c1_s[...]) + 0.4 * t2
        alpha = jnp.where(mc > 0.0, alpha, -1e30)

        cmax = jnp.max(alpha, axis=0, keepdims=True)
        new_max = jnp.maximum(max_s[...], cmax)
        scale = jnp.exp(max_s[...] - new_max)
        w = mc * jnp.exp(alpha - new_max)
        den_s[...] = den_s[...] * scale + jnp.sum(w, axis=0, keepdims=True)
        dg = lax.dot_general(w, xl, (((0,), (0,)), ((), ())),
                             preferred_element_type=jnp.float32)
        for h in range(HEADS):
            num_s[h:h + 1, :] = num_s[h:h + 1, :] * scale[0:1, h:h + 1] + \
                dg[h:h + 1, h * GNN:(h + 1) * GNN]
        max_s[...] = new_max

    @pl.when(s == nch)
    def _final():
        recip = 1.0 / (den_s[...] + 1e-16)
        acc = jnp.zeros((1, GNN), jnp.float32)
        for h in range(HEADS):
            acc = acc + num_s[h:h + 1, :] * recip[0:1, h:h + 1]
        gat = acc * (1.0 / HEADS) + gb_ref[...]
        gi = jnp.dot(gat, Wih_ref[...],
                     preferred_element_type=jnp.float32) + bih_ref[...]
        gh = jnp.dot(hid_ref[...], Whh_ref[...],
                     preferred_element_type=jnp.float32) + bhh_ref[...]
        r = jax.nn.sigmoid(gi[:, 0:RNN] + gh[:, 0:RNN])
        z = jax.nn.sigmoid(gi[:, RNN:2 * RNN] + gh[:, RNN:2 * RNN])
        n = jnp.tanh(gi[:, 2 * RNN:] + r * gh[:, 2 * RNN:])
        out_ref[...] = (1.0 - z) * n + z * hid_ref[...]


def _tc_call(nf, mgrid, hidden, Wenc, benc, Wl, bl, Wr, br, Ad, Wlad, blad,
             gb, Wih, bih, Whh, bhh, interpret=False):
    N = nf.shape[0]
    nch = N // CHUNK
    grid = nch + 1
    chunk_of = lambda s: jnp.minimum(s, nch - 1)

    full = lambda shp: pl.BlockSpec(shp, lambda s: tuple(0 for _ in shp))
    in_specs = [
        pl.BlockSpec((CHUNK, nf.shape[1]), lambda s: (chunk_of(s), 0)),
        full(mgrid.shape),
        full(hidden.shape),
        full(Wenc.shape), full(benc.shape),
        full(Wl.shape), full(bl.shape),
        full(Wr.shape), full(br.shape),
        full(Ad.shape), full(Wlad.shape), full(blad.shape),
        full(gb.shape),
        full(Wih.shape), full(bih.shape),
        full(Whh.shape), full(bhh.shape),
    ]
    return pl.pallas_call(
        _tc_body,
        grid=(grid,),
        in_specs=in_specs,
        out_specs=full((1, RNN)),
        out_shape=jax.ShapeDtypeStruct((1, RNN), jnp.float32),
        scratch_shapes=[
            pltpu.VMEM((1, HEADS * GNN), jnp.float32),
            pltpu.VMEM((1, HEADS), jnp.float32),
            pltpu.VMEM((1, HEADS), jnp.float32),
            pltpu.VMEM((1, HEADS), jnp.float32),
            pltpu.VMEM((HEADS, GNN), jnp.float32),
            pltpu.VMEM((NPAD, 1), jnp.float32),
        ],
        interpret=interpret,
    )(nf, mgrid, hidden, Wenc, benc, Wl, bl, Wr, br, Ad, Wlad, blad, gb,
      Wih, bih, Whh, bhh)


def _tc_sparse_body(rows_ref, ml_ref, nf0_ref, hid_ref, Wenc_ref, benc_ref,
                    Wl_ref, bl_ref, Wr_ref, br_ref, Ad_ref, Wlad_ref,
                    blad_ref, gb_ref, Wih_ref, bih_ref, Whh_ref, bhh_ref,
                    out_ref):
    x0 = jnp.maximum(
        jnp.dot(nf0_ref[0:1, :], Wenc_ref[...],
                preferred_element_type=jnp.float32) + benc_ref[...], 0.0)
    xr0 = jnp.dot(x0, Wr_ref[...],
                  preferred_element_type=jnp.float32) + br_ref[...]
    c1 = jnp.dot(xr0, Ad_ref[...], preferred_element_type=jnp.float32)

    x = jnp.maximum(
        jnp.dot(rows_ref[...], Wenc_ref[...],
                preferred_element_type=jnp.float32) + benc_ref[...], 0.0)
    xl = jnp.dot(x, Wl_ref[...],
                 preferred_element_type=jnp.float32) + bl_ref[...]

    mg = ml_ref[...]
    rep = jnp.broadcast_to(mg[:, None, :], (NSP // GNN, GNN, GNN))
    rep = rep.reshape(NSP, GNN)
    lane = lax.broadcasted_iota(jnp.int32, (NSP, GNN), 1)
    rowm = lax.broadcasted_iota(jnp.int32, (NSP, GNN), 0) & (GNN - 1)
    mc = jnp.sum(jnp.where(lane == rowm, rep, 0.0), axis=1, keepdims=True)

    t1 = jnp.dot(x, Wlad_ref[...],
                 preferred_element_type=jnp.float32) + blad_ref[...]
    v = xl + xr0
    t2 = jnp.dot(jnp.abs(v), Ad_ref[...], preferred_element_type=jnp.float32)
    alpha = 0.6 * (t1 + c1) + 0.4 * t2
    alpha = jnp.where(mc > 0.0, alpha, -1e30)
    amax = jnp.max(alpha, axis=0, keepdims=True)
    w = mc * jnp.exp(alpha - amax)
    den = jnp.sum(w, axis=0, keepdims=True)
    dg = lax.dot_general(w, xl, (((0,), (0,)), ((), ())),
                         preferred_element_type=jnp.float32)
    recip = 1.0 / (den + 1e-16)
    acc = jnp.zeros((1, GNN), jnp.float32)
    for h in range(HEADS):
        acc = acc + dg[h:h + 1, h * GNN:(h + 1) * GNN] * recip[0:1, h:h + 1]
    gat = acc * (1.0 / HEADS) + gb_ref[...]
    gi = jnp.dot(gat, Wih_ref[...],
                 preferred_element_type=jnp.float32) + bih_ref[...]
    gh = jnp.dot(hid_ref[...], Whh_ref[...],
                 preferred_element_type=jnp.float32) + bhh_ref[...]
    r = jax.nn.sigmoid(gi[:, 0:RNN] + gh[:, 0:RNN])
    z = jax.nn.sigmoid(gi[:, RNN:2 * RNN] + gh[:, RNN:2 * RNN])
    n = jnp.tanh(gi[:, 2 * RNN:] + r * gh[:, 2 * RNN:])
    out_ref[...] = (1.0 - z) * n + z * hid_ref[...]


def _tc_sparse_call(rows, mg, nf0, hidden, Wenc, benc, Wl, bl, Wr, br, Ad,
                    Wlad, blad, gb, Wih, bih, Whh, bhh, interpret=False):
    full = lambda shp: pl.BlockSpec(shp, lambda s: tuple(0 for _ in shp))
    in_specs = [
        full(rows.shape), full(mg.shape),
        pl.BlockSpec((8, nf0.shape[1]), lambda s: (0, 0)),
        full(hidden.shape),
        full(Wenc.shape), full(benc.shape), full(Wl.shape), full(bl.shape),
        full(Wr.shape), full(br.shape), full(Ad.shape), full(Wlad.shape),
        full(blad.shape), full(gb.shape), full(Wih.shape), full(bih.shape),
        full(Whh.shape), full(bhh.shape),
    ]
    return pl.pallas_call(
        _tc_sparse_body,
        grid=(1,),
        in_specs=in_specs,
        out_specs=full((1, RNN)),
        out_shape=jax.ShapeDtypeStruct((1, RNN), jnp.float32),
        interpret=interpret,
    )(rows, mg, nf0, hidden, Wenc, benc, Wl, bl, Wr, br, Ad, Wlad, blad,
      gb, Wih, bih, Whh, bhh)


def kernel(node_features, edge_index, hidden_state, W_enc, b_enc, W_l, b_l,
           W_r, b_r, att, gat_bias, W_ih, b_ih, W_hh, b_hh):
    ei = edge_index
    if ei.dtype != jnp.int32:
        ei = ei.astype(jnp.int32)
    mgrid, rows, ml, kc = _sc_multiplicity(ei, node_features)
    Ad = (att[:, :, None] * jnp.eye(HEADS, dtype=att.dtype)[:, None, :]
          ).reshape(HEADS * GNN, HEADS)
    Wlad = W_l @ Ad
    blad = (b_l @ Ad).reshape(1, HEADS)
    row = lambda v: v.reshape(1, -1)
    args = (hidden_state, W_enc, row(b_enc), W_l, row(b_l), W_r, row(b_r),
            Ad, Wlad, blad, row(gat_bias), W_ih, row(b_ih), W_hh, row(b_hh))

    def sparse_path():
        return _tc_sparse_call(rows.reshape(NSP, GNN),
                               ml.reshape(NSP // GNN, GNN),
                               node_features, *args)

    def dense_path():
        return _tc_call(node_features, mgrid, *args)

    return lax.cond(jnp.max(kc) <= CAP_T, sparse_path, dense_path)


# speedup vs baseline: 1.3973x; 1.3973x over previous
import functools

import jax
import jax.numpy as jnp
from jax import lax
from jax.experimental import pallas as pl
from jax.experimental.pallas import tpu as pltpu
from jax.experimental.pallas import tpu_sc as plsc

GNN = 128
RNN = 256
HEADS = 4
CHUNK = 2000
LANES = 16
SC_CORES = 2
SC_SUBCORES = 16
SC_WORKERS = SC_CORES * SC_SUBCORES
MROWS = 80
NPAD = MROWS * GNN


def _sc_multiplicity(edge_index):
    E = edge_index.shape[1]
    tiles = E // 128
    tpw = tiles // SC_WORKERS
    extra = tiles - tpw * SC_WORKERS
    epw = tpw * 128
    mesh = plsc.VectorSubcoreMesh(core_axis_name="c", subcore_axis_name="s")

    @functools.partial(
        pl.kernel,
        out_type=jax.ShapeDtypeStruct((SC_CORES, MROWS, GNN), jnp.float32),
        mesh=mesh,
        scratch_types=[
            pltpu.VMEM((2, epw + 128), jnp.int32),
            pltpu.VMEM((MROWS, GNN), jnp.float32),
            pltpu.VMEM((MROWS,), jnp.int32),
            pltpu.VMEM_SHARED((MROWS, GNN), jnp.float32),
        ],
        compiler_params=pltpu.CompilerParams(needs_layout_passes=False),
    )
    def k(edge_hbm, out_hbm, e_v, m_v, idx_v, shared):
        cid = lax.axis_index("c")
        sid = lax.axis_index("s")
        wid = sid * SC_CORES + cid
        base = wid * epw
        pltpu.sync_copy(edge_hbm.at[:, pl.ds(base, epw)],
                        e_v.at[:, pl.ds(0, epw)])

        @pl.when(wid < extra)
        def _():
            xoff = (SC_WORKERS * tpw + wid) * 128
            pltpu.sync_copy(edge_hbm.at[:, pl.ds(xoff, 128)],
                            e_v.at[:, pl.ds(epw, 128)])

        zeros16 = jnp.zeros((LANES,), jnp.float32)
        iota16 = lax.broadcasted_iota(jnp.int32, (LANES,), 0)

        def zero_body(r, carry):
            for j in range(8):
                m_v[r, pl.ds(j * LANES, LANES)] = zeros16
            return carry

        lax.fori_loop(0, MROWS, zero_body, 0)

        def idx_body(j, carry):
            idx_v[pl.ds(j * LANES, LANES)] = iota16 + j * LANES
            return carry

        lax.fori_loop(0, MROWS // LANES, idx_body, 0)

        @pl.when(sid == 0)
        def _():
            pltpu.sync_copy(m_v, shared)

        plsc.subcore_barrier()

        ones16 = jnp.ones((LANES,), jnp.float32)

        def edge_one(i):
            s16 = e_v[0, pl.ds(i * LANES, LANES)]
            d16 = e_v[1, pl.ds(i * LANES, LANES)]
            plsc.addupdate_scatter(
                m_v, [s16 >> 7, s16 & 127], ones16, mask=d16 == 0)

        plsc.parallel_loop(0, epw // LANES, 1, unroll=8)(edge_one)

        @pl.when(wid < extra)
        def _():
            def edge_fori(i, carry):
                edge_one(i)
                return carry
            lax.fori_loop(epw // LANES, (epw + 128) // LANES, edge_fori, 0)

        pltpu.sync_copy(m_v, shared.at[idx_v], add=True)
        plsc.subcore_barrier()

        @pl.when(sid == 0)
        def _():
            pltpu.sync_copy(shared, out_hbm.at[cid])

    return k(edge_index)


def _tc_body(nf_ref, m_ref, hid_ref, Wenc_ref, benc_ref, Wl_ref, bl_ref,
             Wr_ref, br_ref, Ad_ref, Wlad_ref, blad_ref, gb_ref, Wih_ref,
             bih_ref, Whh_ref, bhh_ref, out_ref,
             xr0_s, c1_s, max_s, den_s, num_s, mcol_s):
    s = pl.program_id(0)
    nch = pl.num_programs(0) - 1

    @pl.when(s == 0)
    def _init():
        max_s[...] = jnp.full_like(max_s, -1e30)
        den_s[...] = jnp.zeros_like(den_s)
        num_s[...] = jnp.zeros_like(num_s)
        mgrid = m_ref[0] + m_ref[1]
        rep = jnp.broadcast_to(mgrid[:, None, :], (MROWS, GNN, GNN))
        rep = rep.reshape(NPAD, GNN)
        lane = lax.broadcasted_iota(jnp.int32, (NPAD, GNN), 1)
        rowm = lax.broadcasted_iota(jnp.int32, (NPAD, GNN), 0) & (GNN - 1)
        mcol = jnp.sum(jnp.where(lane == rowm, rep, 0.0), axis=1, keepdims=True)
        row0 = lax.broadcasted_iota(jnp.int32, (NPAD, 1), 0) == 0
        mcol_s[...] = mcol + jnp.where(row0, 1.0, 0.0)

    @pl.when(s < nch)
    def _scan():
        x = jnp.maximum(
            jnp.dot(nf_ref[...], Wenc_ref[...],
                    preferred_element_type=jnp.float32) + benc_ref[...], 0.0)
        xl = jnp.dot(x, Wl_ref[...],
                     preferred_element_type=jnp.float32) + bl_ref[...]

        @pl.when(s == 0)
        def _():
            xr0 = jnp.dot(x[0:1, :], Wr_ref[...],
                          preferred_element_type=jnp.float32) + br_ref[...]
            xr0_s[...] = xr0
            c1_s[...] = jnp.dot(xr0, Ad_ref[...],
                                preferred_element_type=jnp.float32)

        mc = mcol_s[pl.ds(s * CHUNK, CHUNK), :]

        t1 = jnp.dot(x, Wlad_ref[...],
                     preferred_element_type=jnp.float32) + blad_ref[...]
        v = xl + xr0_s[...]
        t2 = jnp.dot(jnp.abs(v), Ad_ref[...],
                     preferred_element_type=jnp.float32)
        alpha = 0.6 * (t1 + c1_s[...]) + 0.4 * t2
        alpha = jnp.where(mc > 0.0, alpha, -1e30)

        cmax = jnp.max(alpha, axis=0, keepdims=True)
        new_max = jnp.maximum(max_s[...], cmax)
        scale = jnp.exp(max_s[...] - new_max)
        w = mc * jnp.exp(alpha - new_max)
        den_s[...] = den_s[...] * scale + jnp.sum(w, axis=0, keepdims=True)
        dg = lax.dot_general(w, xl, (((0,), (0,)), ((), ())),
                             preferred_element_type=jnp.float32)
        for h in range(HEADS):
            num_s[h:h + 1, :] = num_s[h:h + 1, :] * scale[0:1, h:h + 1] + \
                dg[h:h + 1, h * GNN:(h + 1) * GNN]
        max_s[...] = new_max

    @pl.when(s == nch)
    def _final():
        recip = 1.0 / (den_s[...] + 1e-16)
        acc = jnp.zeros((1, GNN), jnp.float32)
        for h in range(HEADS):
            acc = acc + num_s[h:h + 1, :] * recip[0:1, h:h + 1]
        gat = acc * (1.0 / HEADS) + gb_ref[...]
        gi = jnp.dot(gat, Wih_ref[...],
                     preferred_element_type=jnp.float32) + bih_ref[...]
        gh = jnp.dot(hid_ref[...], Whh_ref[...],
                     preferred_element_type=jnp.float32) + bhh_ref[...]
        r = jax.nn.sigmoid(gi[:, 0:RNN] + gh[:, 0:RNN])
        z = jax.nn.sigmoid(gi[:, RNN:2 * RNN] + gh[:, RNN:2 * RNN])
        n = jnp.tanh(gi[:, 2 * RNN:] + r * gh[:, 2 * RNN:])
        out_ref[...] = (1.0 - z) * n + z * hid_ref[...]


def _tc_call(nf, mgrid, hidden, Wenc, benc, Wl, bl, Wr, br, Ad, Wlad, blad,
             gb, Wih, bih, Whh, bhh, interpret=False):
    N = nf.shape[0]
    nch = N // CHUNK
    grid = nch + 1
    chunk_of = lambda s: jnp.minimum(s, nch - 1)

    full = lambda shp: pl.BlockSpec(shp, lambda s: tuple(0 for _ in shp))
    in_specs = [
        pl.BlockSpec((CHUNK, nf.shape[1]), lambda s: (chunk_of(s), 0)),
        full(mgrid.shape),
        full(hidden.shape),
        full(Wenc.shape), full(benc.shape),
        full(Wl.shape), full(bl.shape),
        full(Wr.shape), full(br.shape),
        full(Ad.shape), full(Wlad.shape), full(blad.shape),
        full(gb.shape),
        full(Wih.shape), full(bih.shape),
        full(Whh.shape), full(bhh.shape),
    ]
    return pl.pallas_call(
        _tc_body,
        grid=(grid,),
        in_specs=in_specs,
        out_specs=full((1, RNN)),
        out_shape=jax.ShapeDtypeStruct((1, RNN), jnp.float32),
        scratch_shapes=[
            pltpu.VMEM((1, HEADS * GNN), jnp.float32),
            pltpu.VMEM((1, HEADS), jnp.float32),
            pltpu.VMEM((1, HEADS), jnp.float32),
            pltpu.VMEM((1, HEADS), jnp.float32),
            pltpu.VMEM((HEADS, GNN), jnp.float32),
            pltpu.VMEM((NPAD, 1), jnp.float32),
        ],
        interpret=interpret,
    )(nf, mgrid, hidden, Wenc, benc, Wl, bl, Wr, br, Ad, Wlad, blad, gb,
      Wih, bih, Whh, bhh)


def kernel(node_features, edge_index, hidden_state, W_enc, b_enc, W_l, b_l,
           W_r, b_r, att, gat_bias, W_ih, b_ih, W_hh, b_hh):
    ei = edge_index
    if ei.dtype != jnp.int32:
        ei = ei.astype(jnp.int32)
    mgrid = _sc_multiplicity(ei)
    Ad = (att[:, :, None] * jnp.eye(HEADS, dtype=att.dtype)[:, None, :]
          ).reshape(HEADS * GNN, HEADS)
    Wlad = W_l @ Ad
    blad = (b_l @ Ad).reshape(1, HEADS)
    row = lambda v: v.reshape(1, -1)
    return _tc_call(node_features, mgrid, hidden_state, W_enc, row(b_enc),
                    W_l, row(b_l), W_r, row(b_r), Ad, Wlad, blad,
                    row(gat_bias), W_ih, row(b_ih), W_hh, row(b_hh))
